# baseline (speedup 1.0000x reference)
import functools

import jax
import jax.numpy as jnp
from jax import lax
from jax.experimental import pallas as pl
from jax.experimental.pallas import tpu as pltpu
from jax.experimental.pallas import tpu_sc as plsc

NC = 2
NS = 16
NW = NC * NS

IDX_ROW = 128
CHUNK_ROWS = 10
CHUNK = IDX_ROW * CHUNK_ROWS


def _gather_body(table_hbm, idx_hbm, out_hbm,
                 idx_v, rows_v, tbuf, isem0, isem1, gsem0, gsem1, osem):
    D = table_hbm.shape[1]
    n_rows = idx_hbm.shape[0]
    rows_per_w = n_rows // NW
    chunks = rows_per_w // CHUNK_ROWS
    isems = (isem0, isem1)
    gsems = (gsem0, gsem1)

    wid = lax.axis_index("s") * NC + lax.axis_index("c")
    row0 = wid * rows_per_w
    iota16 = lax.iota(jnp.int32, 16)

    def idx_dma(c, b):
        pltpu.async_copy(
            idx_hbm.at[pl.ds(row0 + c * CHUNK_ROWS, CHUNK_ROWS)],
            idx_v.at[b], isems[b])

    def idx_start(c, b):
        @pl.when(c < chunks)
        def _():
            idx_dma(c, b)

    def idx_wait(b):
        pltpu.make_async_copy(
            idx_hbm.at[pl.ds(0, CHUNK_ROWS)], idx_v.at[b], isems[b]).wait()

    def fire_gathers(b):
        for j in range(CHUNK_ROWS):
            pltpu.async_copy(
                table_hbm.at[idx_v.at[b].at[j]],
                rows_v.at[b].at[pl.ds(j * IDX_ROW, IDX_ROW)], gsems[b])

    def wait_gathers(b):
        for j in range(CHUNK_ROWS):
            pltpu.make_async_copy(
                table_hbm.at[idx_v.at[b].at[j]],
                rows_v.at[b].at[pl.ds(j * IDX_ROW, IDX_ROW)], gsems[b]).wait()

    def transpose(b):
        rows_buf = rows_v.at[b]

        def d_body(d, carry):
            col = jnp.full((16,), d, jnp.int32)
            for k in range(CHUNK // 16):
                v = plsc.load_gather(rows_buf, [iota16 + (16 * k), col])
                tbuf[d, pl.ds(16 * k, 16)] = v
            return carry

        lax.fori_loop(0, D, d_body, 0)

    def wb_start(c):
        pltpu.async_copy(
            tbuf, out_hbm.at[:, pl.ds((row0 + c * CHUNK_ROWS) * IDX_ROW, CHUNK)],
            osem)

    def wb_wait():
        pltpu.make_async_copy(
            tbuf, out_hbm.at[:, pl.ds(0, CHUNK)], osem).wait()

    idx_dma(0, 0)
    idx_dma(1, 1)
    idx_wait(0)
    fire_gathers(0)
    idx_wait(1)
    fire_gathers(1)
    wait_gathers(0)
    idx_start(2, 0)
    transpose(0)
    wb_start(0)

    def step(g, carry):
        for b, boff in ((0, 0), (1, 1)):
            c = g * 2 + boff
            idx_wait(b)
            fire_gathers(b)
            wait_gathers(1 - b)
            idx_start(c + 1, 1 - b)
            wb_wait()
            transpose(1 - b)
            wb_start(c - 1)
        return carry

    lax.fori_loop(1, chunks // 2, step, 0)

    wait_gathers(1)
    wb_wait()
    transpose(1)
    wb_start(chunks - 1)
    wb_wait()


@functools.partial(jax.jit, static_argnames=())
def kernel(input, embeddings):
    orig_shape = input.shape
    D = embeddings.shape[1]
    flat_idx = input.reshape(-1).astype(jnp.int32)
    B = flat_idx.shape[0]
    idx2d = flat_idx.reshape(B // IDX_ROW, IDX_ROW)

    mesh = plsc.VectorSubcoreMesh(
        core_axis_name="c", subcore_axis_name="s",
        num_cores=NC, num_subcores=NS,
    )
    out_t = pl.kernel(
        _gather_body,
        out_type=jax.ShapeDtypeStruct((D, B), jnp.float32),
        mesh=mesh,
        scratch_types=[
            pltpu.VMEM((2, CHUNK_ROWS, IDX_ROW), jnp.int32),
            pltpu.VMEM((2, CHUNK, D), jnp.float32),
            pltpu.VMEM((D, CHUNK), jnp.float32),
            pltpu.SemaphoreType.DMA,
            pltpu.SemaphoreType.DMA,
            pltpu.SemaphoreType.DMA,
            pltpu.SemaphoreType.DMA,
            pltpu.SemaphoreType.DMA,
        ],
        compiler_params=pltpu.CompilerParams(
            use_tc_tiling_on_sc=False, needs_layout_passes=False),
    )(embeddings, idx2d)
    return out_t.T.reshape(*orig_shape, D)


# speedup vs baseline: 3.9590x; 3.9590x over previous
import functools

import jax
import jax.numpy as jnp
from jax import lax
from jax.experimental import pallas as pl
from jax.experimental.pallas import tpu as pltpu
from jax.experimental.pallas import tpu_sc as plsc

NC = 2
NS = 16
NW = NC * NS

IDX_ROW = 128
CHUNK_ROWS = 4
CHUNK = IDX_ROW * CHUNK_ROWS
CBLK = CHUNK // IDX_ROW


def _gather_body(table_hbm, idx_hbm, out_hbm,
                 idx_v, rows_v, tbuf, isem0, isem1, gsem0, gsem1, osem):
    D = table_hbm.shape[1]
    n_rows = idx_hbm.shape[0]
    rows_per_w = n_rows // NW
    chunks = rows_per_w // CHUNK_ROWS
    isems = (isem0, isem1)
    gsems = (gsem0, gsem1)

    wid = lax.axis_index("s") * NC + lax.axis_index("c")
    row0 = wid * rows_per_w
    iota16 = lax.iota(jnp.int32, 16)

    def idx_dma(c, b):
        pltpu.async_copy(
            idx_hbm.at[pl.ds(row0 + c * CHUNK_ROWS, CHUNK_ROWS)],
            idx_v.at[b], isems[b])

    def idx_start(c, b):
        @pl.when(c < chunks)
        def _():
            idx_dma(c, b)

    def idx_wait(b):
        pltpu.make_async_copy(
            idx_hbm.at[pl.ds(0, CHUNK_ROWS)], idx_v.at[b], isems[b]).wait()

    def fire_gathers(b):
        for j in range(CHUNK_ROWS):
            pltpu.async_copy(
                table_hbm.at[idx_v.at[b].at[j]],
                rows_v.at[b].at[pl.ds(j * IDX_ROW, IDX_ROW)], gsems[b])

    def wait_gathers(b):
        for j in range(CHUNK_ROWS):
            pltpu.make_async_copy(
                table_hbm.at[idx_v.at[b].at[j]],
                rows_v.at[b].at[pl.ds(j * IDX_ROW, IDX_ROW)], gsems[b]).wait()

    def transpose(b):
        rows_buf = rows_v.at[b]

        def d_body(d, carry):
            a = lax.shift_right_logical(d, 3)
            d8 = lax.bitwise_and(d, 7)
            col = jnp.full((16,), d, jnp.int32)
            for cc in range(CBLK):
                for k in range(IDX_ROW // 16):
                    v = plsc.load_gather(
                        rows_buf, [iota16 + (cc * IDX_ROW + 16 * k), col])
                    tbuf[a, cc, d8, pl.ds(16 * k, 16)] = v
            return carry

        lax.fori_loop(0, D, d_body, 0)

    def out_pos(c):
        p0 = row0 * IDX_ROW + c * CHUNK
        jcol = lax.shift_right_logical(p0, 14)
        cblk = lax.shift_right_logical(
            lax.bitwise_and(p0, 16383), 7)
        return jcol, cblk

    def wb_start(c):
        jcol, cblk = out_pos(c)
        pltpu.async_copy(
            tbuf, out_hbm.at[jcol].at[:, pl.ds(cblk, CBLK)], osem)

    def wb_wait():
        pltpu.make_async_copy(
            tbuf, out_hbm.at[0].at[:, pl.ds(0, CBLK)], osem).wait()

    idx_dma(0, 0)
    idx_dma(1, 1)
    idx_wait(0)
    fire_gathers(0)
    idx_wait(1)
    fire_gathers(1)
    wait_gathers(0)
    idx_start(2, 0)
    transpose(0)
    wb_start(0)

    def step(g, carry):
        for b, boff in ((0, 0), (1, 1)):
            c = g * 2 + boff
            idx_wait(b)
            fire_gathers(b)
            wait_gathers(1 - b)
            idx_start(c + 1, 1 - b)
            wb_wait()
            transpose(1 - b)
            wb_start(c - 1)
        return carry

    lax.fori_loop(1, chunks // 2, step, 0)

    wait_gathers(1)
    wb_wait()
    transpose(1)
    wb_start(chunks - 1)
    wb_wait()


@functools.partial(jax.jit, static_argnames=())
def kernel(input, embeddings):
    n_i, n_j = input.shape
    D = embeddings.shape[1]
    flat_idx = input.T.reshape(-1).astype(jnp.int32)
    B = flat_idx.shape[0]
    idx2d = flat_idx.reshape(B // IDX_ROW, IDX_ROW)

    mesh = plsc.VectorSubcoreMesh(
        core_axis_name="c", subcore_axis_name="s",
        num_cores=NC, num_subcores=NS,
    )
    out5 = pl.kernel(
        _gather_body,
        out_type=jax.ShapeDtypeStruct(
            (n_j, D // 8, n_i // IDX_ROW, 8, IDX_ROW), jnp.float32),
        mesh=mesh,
        scratch_types=[
            pltpu.VMEM((2, CHUNK_ROWS, IDX_ROW), jnp.int32),
            pltpu.VMEM((2, CHUNK, D), jnp.float32),
            pltpu.VMEM((D // 8, CBLK, 8, IDX_ROW), jnp.float32),
            pltpu.SemaphoreType.DMA,
            pltpu.SemaphoreType.DMA,
            pltpu.SemaphoreType.DMA,
            pltpu.SemaphoreType.DMA,
            pltpu.SemaphoreType.DMA,
        ],
        compiler_params=pltpu.CompilerParams(
            use_tc_tiling_on_sc=False, needs_layout_passes=False),
    )(embeddings, idx2d)
    return out5.transpose(2, 4, 0, 1, 3).reshape(n_i, n_j, D)
